# baseline (speedup 1.0000x reference)
import functools

import jax
import jax.numpy as jnp
from jax import lax
from jax.experimental import pallas as pl
from jax.experimental.pallas import tpu as pltpu
from jax.experimental.pallas import tpu_sc as plsc

_N = 10000
_E = 320000
_DH = 128
_DP = 16

_NC = 2
_NS = 16
_NW = _NC * _NS
_EW = _E // _NW
_W = 128
_NWIN = 79
_TAIL = _EW - 78 * _W
_NPAD = 10240
_RPT = _NPAD // _NS

_ROWB = 400


def _mesh():
    return plsc.VectorSubcoreMesh(core_axis_name="c", subcore_axis_name="s")


_NEV = 4
_NRB = 2
_LOOPW = 72
_UNROLL = 18


def _idx_start(eix, base, k, ev, si):
    pltpu.async_copy(eix.at[pl.ds(base + k * _W, _W)], ev.at[0], si)
    pltpu.async_copy(eix.at[pl.ds(_E + base + k * _W, _W)], ev.at[1], si)


def _idx_wait(eix, base, k, ev, si):
    pltpu.make_async_copy(eix.at[pl.ds(base + k * _W, _W)], ev.at[0], si).wait()
    pltpu.make_async_copy(eix.at[pl.ds(_E + base + k * _W, _W)], ev.at[1], si).wait()


_DEV = 4
_DPAIRS = 19


def _didx_start(eix, base, k, ev, si):
    pltpu.async_copy(eix.at[pl.ds(_E + base + k * _W, _W)], ev, si)


def _didx_wait(eix, base, k, ev, si):
    pltpu.make_async_copy(eix.at[pl.ds(_E + base + k * _W, _W)], ev, si).wait()


def _deg_body(eix_hbm, zeros_hbm, out_hbm, evs, evt, onesv, acc, sis, sit, sst, sss):
    c = lax.axis_index("c")
    s = lax.axis_index("s")
    w = s * _NC + c
    base = w * _EW
    for j in range(_W // 16):
        onesv[pl.ds(j * 16, 16)] = jnp.ones((16,), jnp.float32)
    _didx_start(eix_hbm, base, 0, evs[0], sis[0])
    _didx_start(eix_hbm, base, 1, evs[1], sis[1])
    pltpu.sync_copy(zeros_hbm.at[pl.ds(s * _RPT, _RPT)],
                    acc.at[pl.ds(s * _RPT, _RPT)])
    plsc.subcore_barrier()

    def _scat_start(be, bs):
        pltpu.async_copy(onesv, acc.at[evs[be]], sss[bs], add=True)

    def _scat_wait(be, bs):
        pltpu.make_async_copy(onesv, acc.at[evs[be]], sss[bs]).wait()

    def _dwindow(k, b, prev_pred, do_prefetch=True):
        _didx_wait(eix_hbm, base, k, evs[b], sis[b])
        if prev_pred is None:
            _scat_wait(b, b % 2)
        else:
            @pl.when(prev_pred)
            def _():
                _scat_wait(b, b % 2)
        _scat_start(b, b % 2)
        if do_prefetch:
            _didx_start(eix_hbm, base, k + 2, evs[(b + 2) % _DEV],
                        sis[(b + 2) % _DEV])

    def body(kk, carry):
        for b in range(_DEV):
            k = kk * _DEV + b
            _dwindow(k, b, (kk > 0) if b < 2 else None)
        return carry

    lax.fori_loop(0, _DPAIRS, body, 0)
    _dwindow(76, 0, None, do_prefetch=False)
    _dwindow(77, 1, None, do_prefetch=False)
    tbase = base + 78 * _W
    pltpu.async_copy(eix_hbm.at[pl.ds(_E + tbase, _TAIL)], evt, sit)
    pltpu.make_async_copy(eix_hbm.at[pl.ds(_E + tbase, _TAIL)], evt, sit).wait()
    _scat_wait(2, 0)
    pltpu.async_copy(onesv.at[pl.ds(0, _TAIL)], acc.at[evt], sst, add=True)
    _scat_wait(3, 1)
    pltpu.make_async_copy(onesv.at[pl.ds(0, _TAIL)], acc.at[evt], sst).wait()
    plsc.subcore_barrier()
    pltpu.sync_copy(acc.at[pl.ds(s * _RPT, _RPT)],
                    out_hbm.at[c, pl.ds(s * _RPT, _RPT)])


_deg_call = pl.kernel(
    _deg_body,
    out_type=jax.ShapeDtypeStruct((_NC, _NPAD), jnp.float32),
    mesh=_mesh(),
    scratch_types=[
        [pltpu.VMEM((_W,), jnp.int32) for _ in range(_DEV)],
        pltpu.VMEM((_TAIL,), jnp.int32),
        pltpu.VMEM((_W,), jnp.float32),
        pltpu.VMEM_SHARED((_NPAD,), jnp.float32),
        [pltpu.SemaphoreType.DMA for _ in range(_DEV)],
        pltpu.SemaphoreType.DMA,
        pltpu.SemaphoreType.DMA,
        [pltpu.SemaphoreType.DMA for _ in range(2)],
    ],
)


def _scat_body(d, g_hbm, eix_hbm, zeros_hbm, out_hbm,
               evs, evt, rows, rowst, acc, sis, sit, sgt, sst, sgs, sss):
    c = lax.axis_index("c")
    s = lax.axis_index("s")
    w = s * _NC + c
    base = w * _EW

    def _g_start(be, br):
        pltpu.async_copy(g_hbm.at[evs[be].at[0]], rows[br], sgs[br])

    def _g_wait(be, br):
        pltpu.make_async_copy(g_hbm.at[evs[be].at[0]], rows[br], sgs[br]).wait()

    def _s_start(be, br):
        pltpu.async_copy(rows[br], acc.at[evs[be].at[1]], sss[br], add=True)

    def _s_wait(be, br):
        pltpu.make_async_copy(rows[br], acc.at[evs[be].at[1]], sss[br]).wait()

    def _window(k, b, prev_pred, do_next=True, do_prefetch=True):
        br = b % _NRB
        _g_wait(b, br)
        _s_start(b, br)
        if prev_pred is None:
            _s_wait((b - 1) % _NEV, (b + 1) % _NRB)
        else:
            @pl.when(prev_pred)
            def _():
                _s_wait((b - 1) % _NEV, (b + 1) % _NRB)
        if do_next:
            _idx_wait(eix_hbm, base, k + 1, evs[(b + 1) % _NEV],
                      sis[(b + 1) % _NEV])
            _g_start((b + 1) % _NEV, (b + 1) % _NRB)
        if do_prefetch:
            _idx_start(eix_hbm, base, k + 2, evs[(b + 2) % _NEV],
                       sis[(b + 2) % _NEV])

    _idx_start(eix_hbm, base, 0, evs[0], sis[0])
    _idx_start(eix_hbm, base, 1, evs[1], sis[1])
    pltpu.sync_copy(zeros_hbm.at[pl.ds(s * _RPT, _RPT)],
                    acc.at[pl.ds(s * _RPT, _RPT)])
    plsc.subcore_barrier()
    _idx_wait(eix_hbm, base, 0, evs[0], sis[0])
    _g_start(0, 0)

    def body(kk, carry):
        for b in range(_NEV):
            k = kk * _NEV + b
            _window(k, b, (kk > 0) if b == 0 else None)
        return carry

    lax.fori_loop(0, _UNROLL, body, 0)
    for k in range(_LOOPW, 76):
        _window(k, k % _NEV, None)
    _window(76, 76 % _NEV, None, do_next=True, do_prefetch=False)
    _window(77, 77 % _NEV, None, do_next=False, do_prefetch=False)
    tbase = base + 78 * _W
    pltpu.async_copy(eix_hbm.at[pl.ds(tbase, _TAIL)], evt.at[0], sit)
    pltpu.async_copy(eix_hbm.at[pl.ds(_E + tbase, _TAIL)], evt.at[1], sit)
    pltpu.make_async_copy(eix_hbm.at[pl.ds(tbase, _TAIL)], evt.at[0], sit).wait()
    pltpu.make_async_copy(eix_hbm.at[pl.ds(_E + tbase, _TAIL)], evt.at[1], sit).wait()
    pltpu.async_copy(g_hbm.at[evt.at[0]], rowst, sgt)
    pltpu.make_async_copy(g_hbm.at[evt.at[0]], rowst, sgt).wait()
    pltpu.async_copy(rowst, acc.at[evt.at[1]], sst, add=True)
    _s_wait(77 % _NEV, 77 % _NRB)
    pltpu.make_async_copy(rowst, acc.at[evt.at[1]], sst).wait()
    plsc.subcore_barrier()
    pltpu.sync_copy(acc.at[pl.ds(s * _RPT, _RPT)],
                    out_hbm.at[c, pl.ds(s * _RPT, _RPT)])


def _make_scatter(d, tc_tiling=True):
    return pl.kernel(
        functools.partial(_scat_body, d),
        out_type=jax.ShapeDtypeStruct((_NC, _NPAD, d), jnp.float32),
        mesh=_mesh(),
        compiler_params=pltpu.CompilerParams(use_tc_tiling_on_sc=tc_tiling),
        scratch_types=[
            [pltpu.VMEM((2, _W), jnp.int32) for _ in range(_NEV)],
            pltpu.VMEM((2, _TAIL), jnp.int32),
            [pltpu.VMEM((_W, d), jnp.float32) for _ in range(_NRB)],
            pltpu.VMEM((_TAIL, d), jnp.float32),
            pltpu.VMEM_SHARED((_NPAD, d), jnp.float32),
            [pltpu.SemaphoreType.DMA for _ in range(_NEV)],
            pltpu.SemaphoreType.DMA,
            pltpu.SemaphoreType.DMA,
            pltpu.SemaphoreType.DMA,
            [pltpu.SemaphoreType.DMA for _ in range(_NRB)],
            [pltpu.SemaphoreType.DMA for _ in range(_NRB)],
        ],
    )


_scat128 = _make_scatter(_DH)
_scat16 = _make_scatter(_DP, tc_tiling=False)


def _tc1a_body(x_ref, w_ref, h_ref):
    h_ref[...] = jnp.dot(x_ref[...], w_ref[...],
                         preferred_element_type=jnp.float32)


def _tc1a(x, w1):
    return pl.pallas_call(
        _tc1a_body,
        out_shape=jax.ShapeDtypeStruct((_N, _DH), jnp.float32),
    )(x, w1)


def _tc1_body(h_ref, deg_ref, g_ref, dinv_ref):
    dinv = lax.rsqrt(deg_ref[...])
    g_ref[...] = h_ref[...] * dinv
    dinv_ref[...] = dinv


def _tc1(h1, deg_col):
    return pl.pallas_call(
        _tc1_body,
        out_shape=[
            jax.ShapeDtypeStruct((_N, _DH), jnp.float32),
            jax.ShapeDtypeStruct((_N, 1), jnp.float32),
        ],
    )(h1, deg_col)


def _tc2_body(acc_ref, g_ref, dinv_ref, b_ref, w_ref, g2_ref):
    acc = acc_ref[0] + acc_ref[1] + g_ref[...]
    o = jnp.maximum(acc * dinv_ref[...] + b_ref[...], 0.0)
    h2 = jnp.dot(o, w_ref[...], preferred_element_type=jnp.float32)
    g2_ref[...] = h2 * dinv_ref[...]


def _tc2(accp, g1, dinv, b1row, w2p):
    return pl.pallas_call(
        _tc2_body,
        grid=(1,),
        in_specs=[
            pl.BlockSpec((_NC, _N, _DH), lambda i: (0, 0, 0)),
            pl.BlockSpec((_N, _DH), lambda i: (0, 0)),
            pl.BlockSpec((_N, 1), lambda i: (0, 0)),
            pl.BlockSpec((1, _DH), lambda i: (0, 0)),
            pl.BlockSpec((_DH, _DP), lambda i: (0, 0)),
        ],
        out_specs=pl.BlockSpec((_N, _DP), lambda i: (0, 0)),
        out_shape=jax.ShapeDtypeStruct((_N, _DP), jnp.float32),
    )(accp, g1, dinv, b1row, w2p)


def _tc3_body(acc_ref, g_ref, dinv_ref, b_ref, o_ref):
    acc = acc_ref[0] + acc_ref[1] + g_ref[...]
    o = acc * dinv_ref[...] + b_ref[...]
    o_ref[...] = o[:, :2]


def _tc3(acc2p, g2, dinv, b2row):
    return pl.pallas_call(
        _tc3_body,
        grid=(1,),
        in_specs=[
            pl.BlockSpec((_NC, _N, _DP), lambda i: (0, 0, 0)),
            pl.BlockSpec((_N, _DP), lambda i: (0, 0)),
            pl.BlockSpec((_N, 1), lambda i: (0, 0)),
            pl.BlockSpec((1, _DP), lambda i: (0, 0)),
        ],
        out_specs=pl.BlockSpec((_N, 2), lambda i: (0, 0)),
        out_shape=jax.ShapeDtypeStruct((_N, 2), jnp.float32),
    )(acc2p, g2, dinv, b2row)


def kernel(x, edge_index, W1, b1, W2, b2):
    eflat = edge_index.reshape(2 * _E)
    zerosd = jnp.zeros((_NPAD,), jnp.float32)
    zeros1 = jnp.zeros((_NPAD, _DH), jnp.float32)
    zeros2 = jnp.zeros((_NPAD, _DP), jnp.float32)
    w2p = jnp.concatenate([W2, jnp.zeros((_DH, _DP - W2.shape[1]), jnp.float32)], axis=1)
    b2p = jnp.concatenate([b2, jnp.zeros((_DP - b2.shape[0],), jnp.float32)]).reshape(1, _DP)
    b1row = b1.reshape(1, _DH)

    h1 = _tc1a(x, W1)
    dp = _deg_call(eflat, zerosd)
    deg_col = (dp[0, :_N] + dp[1, :_N] + 1.0).reshape(_N, 1)

    g1, dinv = _tc1(h1, deg_col)
    accp = _scat128(g1, eflat, zeros1)
    g2 = _tc2(accp, g1, dinv, b1row, w2p)
    acc2p = _scat16(g2, eflat, zeros2)
    return _tc3(acc2p, g2, dinv, b2p)


# speedup vs baseline: 1.0040x; 1.0040x over previous
import functools

import jax
import jax.numpy as jnp
from jax import lax
from jax.experimental import pallas as pl
from jax.experimental.pallas import tpu as pltpu
from jax.experimental.pallas import tpu_sc as plsc

_N = 10000
_E = 320000
_DH = 128
_DP = 16

_NC = 2
_NS = 16
_NW = _NC * _NS
_EW = _E // _NW
_W = 128
_NWIN = 79
_TAIL = _EW - 78 * _W
_NPAD = 10240
_RPT = _NPAD // _NS

_ROWB = 400


def _mesh():
    return plsc.VectorSubcoreMesh(core_axis_name="c", subcore_axis_name="s")


_NEV = 4
_NRB = 2
_LOOPW = 72
_UNROLL = 18


def _idx_start(eix, base, k, ev, si):
    pltpu.async_copy(eix.at[pl.ds(base + k * _W, _W)], ev.at[0], si)
    pltpu.async_copy(eix.at[pl.ds(_E + base + k * _W, _W)], ev.at[1], si)


def _idx_wait(eix, base, k, ev, si):
    pltpu.make_async_copy(eix.at[pl.ds(base + k * _W, _W)], ev.at[0], si).wait()
    pltpu.make_async_copy(eix.at[pl.ds(_E + base + k * _W, _W)], ev.at[1], si).wait()


_DEV = 4
_DPAIRS = 19


def _didx_start(eix, base, k, ev, si):
    pltpu.async_copy(eix.at[pl.ds(_E + base + k * _W, _W)], ev, si)


def _didx_wait(eix, base, k, ev, si):
    pltpu.make_async_copy(eix.at[pl.ds(_E + base + k * _W, _W)], ev, si).wait()


def _deg_body(eix_hbm, zeros_hbm, out_hbm, evs, evt, onesv, acc, sis, sit, sst, sss):
    c = lax.axis_index("c")
    s = lax.axis_index("s")
    w = s * _NC + c
    base = w * _EW
    for j in range(_W // 16):
        onesv[pl.ds(j * 16, 16)] = jnp.ones((16,), jnp.float32)
    _didx_start(eix_hbm, base, 0, evs[0], sis[0])
    _didx_start(eix_hbm, base, 1, evs[1], sis[1])
    pltpu.sync_copy(zeros_hbm.at[pl.ds(s * _RPT, _RPT)],
                    acc.at[pl.ds(s * _RPT, _RPT)])
    plsc.subcore_barrier()

    def _scat_start(be, bs):
        pltpu.async_copy(onesv, acc.at[evs[be]], sss[bs], add=True)

    def _scat_wait(be, bs):
        pltpu.make_async_copy(onesv, acc.at[evs[be]], sss[bs]).wait()

    def _dwindow(k, b, prev_pred, do_prefetch=True):
        _didx_wait(eix_hbm, base, k, evs[b], sis[b])
        if prev_pred is None:
            _scat_wait(b, b % 2)
        else:
            @pl.when(prev_pred)
            def _():
                _scat_wait(b, b % 2)
        _scat_start(b, b % 2)
        if do_prefetch:
            _didx_start(eix_hbm, base, k + 2, evs[(b + 2) % _DEV],
                        sis[(b + 2) % _DEV])

    def body(kk, carry):
        for b in range(_DEV):
            k = kk * _DEV + b
            _dwindow(k, b, (kk > 0) if b < 2 else None)
        return carry

    lax.fori_loop(0, _DPAIRS, body, 0)
    _dwindow(76, 0, None, do_prefetch=False)
    _dwindow(77, 1, None, do_prefetch=False)
    tbase = base + 78 * _W
    pltpu.async_copy(eix_hbm.at[pl.ds(_E + tbase, _TAIL)], evt, sit)
    pltpu.make_async_copy(eix_hbm.at[pl.ds(_E + tbase, _TAIL)], evt, sit).wait()
    _scat_wait(2, 0)
    pltpu.async_copy(onesv.at[pl.ds(0, _TAIL)], acc.at[evt], sst, add=True)
    _scat_wait(3, 1)
    pltpu.make_async_copy(onesv.at[pl.ds(0, _TAIL)], acc.at[evt], sst).wait()
    plsc.subcore_barrier()
    pltpu.sync_copy(acc.at[pl.ds(s * _RPT, _RPT)],
                    out_hbm.at[c, pl.ds(s * _RPT, _RPT)])


_deg_call = pl.kernel(
    _deg_body,
    out_type=jax.ShapeDtypeStruct((_NC, _NPAD), jnp.float32),
    mesh=_mesh(),
    scratch_types=[
        [pltpu.VMEM((_W,), jnp.int32) for _ in range(_DEV)],
        pltpu.VMEM((_TAIL,), jnp.int32),
        pltpu.VMEM((_W,), jnp.float32),
        pltpu.VMEM_SHARED((_NPAD,), jnp.float32),
        [pltpu.SemaphoreType.DMA for _ in range(_DEV)],
        pltpu.SemaphoreType.DMA,
        pltpu.SemaphoreType.DMA,
        [pltpu.SemaphoreType.DMA for _ in range(2)],
    ],
)


def _scat_body(d, g_hbm, eix_hbm, zeros_hbm, out_hbm,
               evs, evt, rows, rowst, acc, sis, sit, sgt, sst, sgs, sss):
    c = lax.axis_index("c")
    s = lax.axis_index("s")
    w = s * _NC + c
    base = w * _EW

    def _g_start(be, br):
        pltpu.async_copy(g_hbm.at[evs[be].at[0]], rows[br], sgs[br])

    def _g_wait(be, br):
        pltpu.make_async_copy(g_hbm.at[evs[be].at[0]], rows[br], sgs[br]).wait()

    def _s_start(be, br):
        pltpu.async_copy(rows[br], acc.at[evs[be].at[1]], sss[br], add=True)

    def _s_wait(be, br):
        pltpu.make_async_copy(rows[br], acc.at[evs[be].at[1]], sss[br]).wait()

    def _window(k, b, prev_pred, do_next=True, do_prefetch=True):
        br = b % _NRB
        _g_wait(b, br)
        _s_start(b, br)
        if prev_pred is None:
            _s_wait((b - 1) % _NEV, (b + 1) % _NRB)
        else:
            @pl.when(prev_pred)
            def _():
                _s_wait((b - 1) % _NEV, (b + 1) % _NRB)
        if do_next:
            _idx_wait(eix_hbm, base, k + 1, evs[(b + 1) % _NEV],
                      sis[(b + 1) % _NEV])
            _g_start((b + 1) % _NEV, (b + 1) % _NRB)
        if do_prefetch:
            _idx_start(eix_hbm, base, k + 2, evs[(b + 2) % _NEV],
                       sis[(b + 2) % _NEV])

    _idx_start(eix_hbm, base, 0, evs[0], sis[0])
    _idx_start(eix_hbm, base, 1, evs[1], sis[1])
    pltpu.sync_copy(zeros_hbm.at[pl.ds(s * _RPT, _RPT)],
                    acc.at[pl.ds(s * _RPT, _RPT)])
    plsc.subcore_barrier()
    _idx_wait(eix_hbm, base, 0, evs[0], sis[0])
    _g_start(0, 0)

    def body(kk, carry):
        for b in range(_NEV):
            k = kk * _NEV + b
            _window(k, b, (kk > 0) if b == 0 else None)
        return carry

    lax.fori_loop(0, _UNROLL, body, 0)
    for k in range(_LOOPW, 76):
        _window(k, k % _NEV, None)
    _window(76, 76 % _NEV, None, do_next=True, do_prefetch=False)
    _window(77, 77 % _NEV, None, do_next=False, do_prefetch=False)
    tbase = base + 78 * _W
    pltpu.async_copy(eix_hbm.at[pl.ds(tbase, _TAIL)], evt.at[0], sit)
    pltpu.async_copy(eix_hbm.at[pl.ds(_E + tbase, _TAIL)], evt.at[1], sit)
    pltpu.make_async_copy(eix_hbm.at[pl.ds(tbase, _TAIL)], evt.at[0], sit).wait()
    pltpu.make_async_copy(eix_hbm.at[pl.ds(_E + tbase, _TAIL)], evt.at[1], sit).wait()
    pltpu.async_copy(g_hbm.at[evt.at[0]], rowst, sgt)
    pltpu.make_async_copy(g_hbm.at[evt.at[0]], rowst, sgt).wait()
    pltpu.async_copy(rowst, acc.at[evt.at[1]], sst, add=True)
    _s_wait(77 % _NEV, 77 % _NRB)
    pltpu.make_async_copy(rowst, acc.at[evt.at[1]], sst).wait()
    plsc.subcore_barrier()
    pltpu.sync_copy(acc.at[pl.ds(s * _RPT, _RPT)],
                    out_hbm.at[c, pl.ds(s * _RPT, _RPT)])


def _make_scatter(d, tc_tiling=True):
    return pl.kernel(
        functools.partial(_scat_body, d),
        out_type=jax.ShapeDtypeStruct((_NC, _NPAD, d), jnp.float32),
        mesh=_mesh(),
        compiler_params=pltpu.CompilerParams(use_tc_tiling_on_sc=tc_tiling),
        scratch_types=[
            [pltpu.VMEM((2, _W), jnp.int32) for _ in range(_NEV)],
            pltpu.VMEM((2, _TAIL), jnp.int32),
            [pltpu.VMEM((_W, d), jnp.float32) for _ in range(_NRB)],
            pltpu.VMEM((_TAIL, d), jnp.float32),
            pltpu.VMEM_SHARED((_NPAD, d), jnp.float32),
            [pltpu.SemaphoreType.DMA for _ in range(_NEV)],
            pltpu.SemaphoreType.DMA,
            pltpu.SemaphoreType.DMA,
            pltpu.SemaphoreType.DMA,
            [pltpu.SemaphoreType.DMA for _ in range(_NRB)],
            [pltpu.SemaphoreType.DMA for _ in range(_NRB)],
        ],
    )


_scat128 = _make_scatter(_DH)
_scat16 = _make_scatter(_DP, tc_tiling=False)


def _tc1_body(x_ref, w_ref, deg_ref, g_ref, dinv_ref):
    dinv = lax.rsqrt(deg_ref[...])
    h = jnp.dot(x_ref[...], w_ref[...], preferred_element_type=jnp.float32)
    g_ref[...] = h * dinv
    dinv_ref[...] = dinv


def _tc1(x, w1, deg_col):
    return pl.pallas_call(
        _tc1_body,
        out_shape=[
            jax.ShapeDtypeStruct((_N, _DH), jnp.float32),
            jax.ShapeDtypeStruct((_N, 1), jnp.float32),
        ],
    )(x, w1, deg_col)


def _tc2_body(acc_ref, g_ref, dinv_ref, b_ref, w_ref, g2_ref):
    acc = acc_ref[0] + acc_ref[1] + g_ref[...]
    o = jnp.maximum(acc * dinv_ref[...] + b_ref[...], 0.0)
    h2 = jnp.dot(o, w_ref[...], preferred_element_type=jnp.float32)
    g2_ref[...] = h2 * dinv_ref[...]


def _tc2(accp, g1, dinv, b1row, w2p):
    return pl.pallas_call(
        _tc2_body,
        grid=(1,),
        in_specs=[
            pl.BlockSpec((_NC, _N, _DH), lambda i: (0, 0, 0)),
            pl.BlockSpec((_N, _DH), lambda i: (0, 0)),
            pl.BlockSpec((_N, 1), lambda i: (0, 0)),
            pl.BlockSpec((1, _DH), lambda i: (0, 0)),
            pl.BlockSpec((_DH, _DP), lambda i: (0, 0)),
        ],
        out_specs=pl.BlockSpec((_N, _DP), lambda i: (0, 0)),
        out_shape=jax.ShapeDtypeStruct((_N, _DP), jnp.float32),
    )(accp, g1, dinv, b1row, w2p)


def _tc3_body(acc_ref, g_ref, dinv_ref, b_ref, o_ref):
    acc = acc_ref[0] + acc_ref[1] + g_ref[...]
    o = acc * dinv_ref[...] + b_ref[...]
    o_ref[...] = o[:, :2]


def _tc3(acc2p, g2, dinv, b2row):
    return pl.pallas_call(
        _tc3_body,
        grid=(1,),
        in_specs=[
            pl.BlockSpec((_NC, _N, _DP), lambda i: (0, 0, 0)),
            pl.BlockSpec((_N, _DP), lambda i: (0, 0)),
            pl.BlockSpec((_N, 1), lambda i: (0, 0)),
            pl.BlockSpec((1, _DP), lambda i: (0, 0)),
        ],
        out_specs=pl.BlockSpec((_N, 2), lambda i: (0, 0)),
        out_shape=jax.ShapeDtypeStruct((_N, 2), jnp.float32),
    )(acc2p, g2, dinv, b2row)


def kernel(x, edge_index, W1, b1, W2, b2):
    eflat = edge_index.reshape(2 * _E)
    zerosd = jnp.zeros((_NPAD,), jnp.float32)
    zeros1 = jnp.zeros((_NPAD, _DH), jnp.float32)
    zeros2 = jnp.zeros((_NPAD, _DP), jnp.float32)
    w2p = jnp.concatenate([W2, jnp.zeros((_DH, _DP - W2.shape[1]), jnp.float32)], axis=1)
    b2p = jnp.concatenate([b2, jnp.zeros((_DP - b2.shape[0],), jnp.float32)]).reshape(1, _DP)
    b1row = b1.reshape(1, _DH)

    dp = _deg_call(eflat, zerosd)
    deg_col = (dp[0, :_N] + dp[1, :_N] + 1.0).reshape(_N, 1)

    g1, dinv = _tc1(x, W1, deg_col)
    accp = _scat128(g1, eflat, zeros1)
    g2 = _tc2(accp, g1, dinv, b1row, w2p)
    acc2p = _scat16(g2, eflat, zeros2)
    return _tc3(acc2p, g2, dinv, b2p)
